# Initial kernel scaffold; baseline (speedup 1.0000x reference)
import jax
import jax.numpy as jnp
from jax.experimental import pallas as pl


def kernel(types, args, table):
    raise NotImplementedError("write your pallas kernel here")


# speedup vs baseline: 5.7903x; 5.7903x over previous
import functools

import jax
import jax.numpy as jnp
from jax import lax
from jax.experimental import pallas as pl
from jax.experimental.pallas import tpu as pltpu
from jax.experimental.pallas import tpu_sc as plsc

INT_PO_TYPE = 1
INT_NE_TYPE = 2
D = 16
CHUNK = 2048


def kernel(types, args, table):
    B, L = types.shape
    N = B * L
    t_flat = types.reshape(N)
    a_flat = args.reshape(N)

    info = plsc.get_sparse_core_info()
    NC, NS = info.num_cores, info.num_subcores
    NW = NC * NS
    per_w = N // NW
    assert per_w * NW == N and per_w % CHUNK == 0
    n_ch = per_w // CHUNK

    mesh = plsc.VectorSubcoreMesh(core_axis_name="c", subcore_axis_name="s")

    @functools.partial(
        pl.kernel,
        mesh=mesh,
        out_type=jax.ShapeDtypeStruct((N, D), jnp.float32),
        compiler_params=pltpu.CompilerParams(
            use_tc_tiling_on_sc=False, needs_layout_passes=False
        ),
        scratch_types=[
            pltpu.VMEM((CHUNK,), jnp.int32),
            pltpu.VMEM((CHUNK,), jnp.float32),
            pltpu.VMEM((CHUNK, D), jnp.float32),
            pltpu.SemaphoreType.DMA,
        ],
    )
    def emb_kernel(t_hbm, a_hbm, tab_hbm, out_hbm, t_v, a_v, rows_v, sem):
        wid = lax.axis_index("s") * NC + lax.axis_index("c")
        base_w = wid * per_w

        def chunk_body(ci, carry):
            base = base_w + ci * CHUNK
            pltpu.sync_copy(t_hbm.at[pl.ds(base, CHUNK)], t_v)
            pltpu.sync_copy(a_hbm.at[pl.ds(base, CHUNK)], a_v)
            pltpu.async_copy(tab_hbm.at[t_v], rows_v, sem).wait()

            def fix_body(j, c):
                t = t_v[pl.ds(j * 16, 16)]
                a = a_v[pl.ds(j * 16, 16)]
                m = (t == INT_PO_TYPE) | (t == INT_NE_TYPE)
                ridx = j * 16 + lax.iota(jnp.int32, 16)
                cidx = jnp.full((16,), D - 1, jnp.int32)
                plsc.store_scatter(rows_v, [ridx, cidx], a, mask=m)
                return c

            lax.fori_loop(0, CHUNK // 16, fix_body, 0)
            pltpu.sync_copy(rows_v, out_hbm.at[pl.ds(base, CHUNK)])
            return carry

        lax.fori_loop(0, n_ch, chunk_body, 0)

    out = emb_kernel(t_flat, a_flat, table)
    return out.reshape(B, L, D)


# speedup vs baseline: 5.9291x; 1.0240x over previous
import functools

import jax
import jax.numpy as jnp
from jax import lax
from jax.experimental import pallas as pl
from jax.experimental.pallas import tpu as pltpu
from jax.experimental.pallas import tpu_sc as plsc

INT_PO_TYPE = 1
INT_NE_TYPE = 2
D = 16
CHUNK = 2048
NBUF = 3


def kernel(types, args, table):
    B, L = types.shape
    N = B * L
    t_flat = types.reshape(N)
    a_flat = args.reshape(N)

    info = plsc.get_sparse_core_info()
    NC, NS = info.num_cores, info.num_subcores
    NW = NC * NS
    per_w = N // NW
    assert per_w * NW == N and per_w % CHUNK == 0
    n_ch = per_w // CHUNK

    mesh = plsc.VectorSubcoreMesh(core_axis_name="c", subcore_axis_name="s")

    @functools.partial(
        pl.kernel,
        mesh=mesh,
        out_type=jax.ShapeDtypeStruct((N, D), jnp.float32),
        compiler_params=pltpu.CompilerParams(
            use_tc_tiling_on_sc=False, needs_layout_passes=False
        ),
        scratch_types=[
            [pltpu.VMEM((CHUNK,), jnp.int32) for _ in range(NBUF)],
            [pltpu.VMEM((CHUNK,), jnp.float32) for _ in range(NBUF)],
            [pltpu.VMEM((CHUNK, D), jnp.float32) for _ in range(NBUF)],
            [pltpu.SemaphoreType.DMA for _ in range(NBUF)],
            [pltpu.SemaphoreType.DMA for _ in range(NBUF)],
            [pltpu.SemaphoreType.DMA for _ in range(NBUF)],
            [pltpu.SemaphoreType.DMA for _ in range(NBUF)],
        ],
    )
    def emb_kernel(t_hbm, a_hbm, tab_hbm, out_hbm,
                   t_v, a_v, rows_v, tsem, asem, gsem, wsem):
        wid = lax.axis_index("s") * NC + lax.axis_index("c")
        base_w = wid * per_w

        def start_load(ci, s):
            base = base_w + ci * CHUNK
            pltpu.async_copy(t_hbm.at[pl.ds(base, CHUNK)], t_v[s], tsem[s])
            pltpu.async_copy(a_hbm.at[pl.ds(base, CHUNK)], a_v[s], asem[s])

        def wait_load(s):
            pltpu.make_async_copy(t_hbm.at[pl.ds(0, CHUNK)], t_v[s], tsem[s]).wait()
            pltpu.make_async_copy(a_hbm.at[pl.ds(0, CHUNK)], a_v[s], asem[s]).wait()

        def start_gather(s):
            pltpu.async_copy(tab_hbm.at[t_v[s]], rows_v[s], gsem[s])

        def wait_gather(s):
            pltpu.make_async_copy(tab_hbm.at[t_v[s]], rows_v[s], gsem[s]).wait()

        def start_write(ci, s):
            base = base_w + ci * CHUNK
            pltpu.async_copy(rows_v[s], out_hbm.at[pl.ds(base, CHUNK)], wsem[s])

        def wait_write(s):
            pltpu.make_async_copy(
                rows_v[s], out_hbm.at[pl.ds(0, CHUNK)], wsem[s]).wait()

        def fixup(s):
            rows = rows_v[s]
            tv, av = t_v[s], a_v[s]

            def fix_body(j, c):
                t = tv[pl.ds(j * 16, 16)]
                a = av[pl.ds(j * 16, 16)]
                m = (t == INT_PO_TYPE) | (t == INT_NE_TYPE)
                ridx = j * 16 + lax.iota(jnp.int32, 16)
                cidx = jnp.full((16,), D - 1, jnp.int32)
                plsc.store_scatter(rows, [ridx, cidx], a, mask=m)
                return c

            lax.fori_loop(0, CHUNK // 16, fix_body, 0)

        start_load(0, 0)
        start_load(1, 1)
        wait_load(0)
        start_gather(0)
        for ci in range(n_ch):
            s = ci % NBUF
            if ci + 2 < n_ch:
                s2 = (ci + 2) % NBUF
                if ci >= 1:
                    wait_write(s2)
                start_load(ci + 2, s2)
            if ci + 1 < n_ch:
                s1 = (ci + 1) % NBUF
                wait_load(s1)
                start_gather(s1)
            wait_gather(s)
            fixup(s)
            start_write(ci, s)
        for k in range(min(NBUF, n_ch)):
            wait_write((n_ch - 1 - k) % NBUF)

    out = emb_kernel(t_flat, a_flat, table)
    return out.reshape(B, L, D)
